# baseline (speedup 1.0000x reference)
import functools

import jax
import jax.numpy as jnp
from jax import lax
from jax.experimental import pallas as pl
from jax.experimental.pallas import tpu as pltpu
from jax.experimental.pallas import tpu_sc as plsc

VOCAB = 1000000
D = 64
NNEG = 5
B = 16384
L = 16
NC = 2
NS = 16
NW = NC * NS
EPW = B // NW
SUB = 128
NSUB = EPW // SUB
SCORES = 1 + NNEG


def _sc_gather_dot(iword, owords, nwords2d, ivw, ovw):
  mesh = plsc.VectorSubcoreMesh(
      core_axis_name="c", subcore_axis_name="s",
      num_cores=NC, num_subcores=NS)

  @functools.partial(
      pl.kernel,
      out_type=jax.ShapeDtypeStruct((B * SCORES, L), jnp.float32),
      mesh=mesh,
      compiler_params=pltpu.CompilerParams(use_tc_tiling_on_sc=False),
      scratch_types=[
          pltpu.VMEM((SUB,), jnp.int32),
          pltpu.VMEM((SUB,), jnp.int32),
          [pltpu.VMEM((SUB,), jnp.int32) for _ in range(NNEG)],
          pltpu.VMEM((SUB, D), jnp.float32),
          pltpu.VMEM((SUB, D), jnp.float32),
          pltpu.VMEM((SUB * NNEG, D), jnp.float32),
          pltpu.VMEM((SUB * SCORES, L), jnp.float32),
          pltpu.SemaphoreType.DMA,
      ],
  )
  def k(iw_hbm, ow_hbm, nw_hbm, ivw_hbm, ovw_hbm, out_hbm,
        iw_idx, ow_idx, nw_idx, iv_rows, ov_rows, nv_rows, out_buf, sem):
    wid = lax.axis_index("s") * NC + lax.axis_index("c")
    for c in range(NSUB):
      off = wid * EPW + c * SUB
      pltpu.sync_copy(iw_hbm.at[pl.ds(off, SUB)], iw_idx)
      pltpu.sync_copy(ow_hbm.at[pl.ds(off, SUB)], ow_idx)
      for j in range(NNEG):
        pltpu.sync_copy(nw_hbm.at[pl.ds(off * NNEG + j * SUB, SUB)], nw_idx[j])
      h_iv = pltpu.async_copy(ivw_hbm.at[iw_idx], iv_rows, sem)
      h_ov = pltpu.async_copy(ovw_hbm.at[ow_idx], ov_rows, sem)
      h_nv = [
          pltpu.async_copy(ovw_hbm.at[nw_idx[j]],
                           nv_rows.at[pl.ds(j * SUB, SUB)], sem)
          for j in range(NNEG)
      ]
      h_iv.wait()
      h_ov.wait()
      for h in h_nv:
        h.wait()

      def elem(b, carry):
        iv = [iv_rows[b, pl.ds(L * j, L)] for j in range(D // L)]
        acc = iv[0] * ov_rows[b, pl.ds(0, L)]
        for j in range(1, D // L):
          acc += iv[j] * ov_rows[b, pl.ds(L * j, L)]
        out_buf[b * SCORES, :] = acc
        for k2 in range(NNEG):
          r = b * NNEG + k2
          nacc = iv[0] * nv_rows[r, pl.ds(0, L)]
          for j in range(1, D // L):
            nacc += iv[j] * nv_rows[r, pl.ds(L * j, L)]
          out_buf[b * SCORES + 1 + k2, :] = nacc
        return carry

      lax.fori_loop(0, SUB, elem, 0)
      pltpu.sync_copy(out_buf, out_hbm.at[pl.ds(off * SCORES, SUB * SCORES)])

  return k(iword, owords, nwords2d, ivw, ovw)


def _tc_reduce_loss(partials2d):
  rows, lanes = partials2d.shape
  groups_per_row = lanes // L

  def body(x_ref, o_ref):
    x = x_ref[...]
    gi = lax.broadcasted_iota(jnp.int32, (lanes, groups_per_row), 0)
    gj = lax.broadcasted_iota(jnp.int32, (lanes, groups_per_row), 1)
    g = (gi // L == gj).astype(jnp.float32)
    s = jax.lax.dot(x, g, preferred_element_type=jnp.float32)
    ri = lax.broadcasted_iota(jnp.int32, (rows, groups_per_row), 0)
    ci = lax.broadcasted_iota(jnp.int32, (rows, groups_per_row), 1)
    gid = ri * groups_per_row + ci
    pos = (gid % SCORES) == 0
    z = jnp.where(pos, s, -s)
    loss = jnp.minimum(z, 0.0) - jnp.log(1.0 + jnp.exp(-jnp.abs(z)))
    o_ref[...] = jnp.full((1, 1), 0.0, jnp.float32) - jnp.sum(loss) / B

  return pl.pallas_call(
      body,
      out_shape=jax.ShapeDtypeStruct((1, 1), jnp.float32),
  )(partials2d)


def kernel(iword, owords, ivec_weight, ovec_weight):
  iword = iword.astype(jnp.int32)
  owords = owords.astype(jnp.int32)
  nwords = jax.random.randint(
      jax.random.key(1), (B, NNEG), 0, VOCAB - 1).astype(jnp.int32)
  partials = _sc_gather_dot(iword, owords, nwords.reshape(B * NNEG),
                            ivec_weight, ovec_weight)
  partials2d = partials.reshape(B * SCORES * L // 128, 128)
  out = _tc_reduce_loss(partials2d)
  return out.reshape(())
